# baseline (speedup 1.0000x reference)
import functools

import jax
import jax.numpy as jnp
from jax import lax
from jax.experimental import pallas as pl
from jax.experimental.pallas import tpu as pltpu
from jax.experimental.pallas import tpu_sc as plsc

D_MODEL = 1024
MAX_LEN = 8192
B_TOTAL = 4 * 8192
NUM_WORKERS = 32
B_PER_W = B_TOTAL // NUM_WORKERS
NB = 8
CHUNK = 8
DEFER = 4
NCHUNKS = B_PER_W // CHUNK
NROUNDS = NCHUNKS // NB


def _sc_gather(pe, idx3):
    mesh = plsc.VectorSubcoreMesh(core_axis_name="c", subcore_axis_name="s")
    num_cores = mesh.num_cores

    @functools.partial(
        pl.kernel,
        out_type=jax.ShapeDtypeStruct((B_TOTAL, D_MODEL), jnp.float32),
        mesh=mesh,
        compiler_params=pltpu.CompilerParams(use_tc_tiling_on_sc=False),
        scratch_types=[
            pltpu.VMEM((NCHUNKS, CHUNK), jnp.int32),
            pltpu.VMEM((NB, CHUNK, D_MODEL), jnp.float32),
            [pltpu.SemaphoreType.DMA] * NB,
            [pltpu.SemaphoreType.DMA] * NB,
        ],
    )
    def k(pe_hbm, idx_hbm, out_hbm, idx_v, bufs, gsems, wsems):
        wid = lax.axis_index("s") * num_cores + lax.axis_index("c")
        base = wid * B_PER_W
        pltpu.sync_copy(idx_hbm.at[wid], idx_v)

        def gather(g, s):
            return pltpu.async_copy(pe_hbm.at[idx_v.at[g]], bufs.at[s], gsems[s])

        def wait_gather(g, s):
            pltpu.make_async_copy(
                pe_hbm.at[idx_v.at[g]], bufs.at[s], gsems[s]).wait()

        def write(g, s):
            return pltpu.async_copy(
                bufs.at[s], out_hbm.at[pl.ds(base + g * CHUNK, CHUNK)], wsems[s])

        def wait_write(g, s):
            pltpu.make_async_copy(
                bufs.at[s],
                out_hbm.at[pl.ds(base + g * CHUNK, CHUNK)], wsems[s]).wait()

        for s in range(NB):
            gather(s, s)

        def body(t, carry):
            g0 = NB * t
            for s in range(NB):
                g = g0 + s
                wait_gather(g, s)
                write(g, s)
                h = g - DEFER

                @pl.when(jnp.logical_and(h >= 0, h + NB < NCHUNKS))
                def _():
                    hs = (s - DEFER) % NB
                    wait_write(h, hs)
                    gather(h + NB, hs)

            return carry

        lax.fori_loop(0, NROUNDS, body, 0)

        for s in range(NB):
            wait_write(NCHUNKS - NB + s, s)

    return k(pe, idx3)


def kernel(positions, pe):
    idx3 = positions.reshape(NUM_WORKERS, NCHUNKS, CHUNK).astype(jnp.int32)
    out = _sc_gather(pe, idx3)
    return out.reshape(positions.shape + (D_MODEL,))


# speedup vs baseline: 2.3998x; 2.3998x over previous
import functools

import jax
import jax.numpy as jnp
from jax import lax
from jax.experimental import pallas as pl
from jax.experimental.pallas import tpu as pltpu
from jax.experimental.pallas import tpu_sc as plsc

D_MODEL = 1024
MAX_LEN = 8192
B_TOTAL = 4 * 8192
NUM_WORKERS = 32
B_PER_W = B_TOTAL // NUM_WORKERS
NB = 4
CHUNK = 16
DEFER = 2
NCHUNKS = B_PER_W // CHUNK
NROUNDS = NCHUNKS // NB


def _sc_gather(pe, idx3):
    mesh = plsc.VectorSubcoreMesh(core_axis_name="c", subcore_axis_name="s")
    num_cores = mesh.num_cores

    @functools.partial(
        pl.kernel,
        out_type=jax.ShapeDtypeStruct((B_TOTAL, D_MODEL), jnp.float32),
        mesh=mesh,
        scratch_types=[
            pltpu.VMEM((NCHUNKS, CHUNK), jnp.int32),
            pltpu.VMEM((NB, CHUNK, D_MODEL), jnp.float32),
            [pltpu.SemaphoreType.DMA] * NB,
            [pltpu.SemaphoreType.DMA] * NB,
        ],
    )
    def k(pe_hbm, idx_hbm, out_hbm, idx_v, bufs, gsems, wsems):
        wid = lax.axis_index("s") * num_cores + lax.axis_index("c")
        base = wid * B_PER_W
        pltpu.sync_copy(idx_hbm.at[wid], idx_v)

        def gather(g, s):
            return pltpu.async_copy(pe_hbm.at[idx_v.at[g]], bufs.at[s], gsems[s])

        def wait_gather(g, s):
            pltpu.make_async_copy(
                pe_hbm.at[idx_v.at[g]], bufs.at[s], gsems[s]).wait()

        def write(g, s):
            return pltpu.async_copy(
                bufs.at[s], out_hbm.at[pl.ds(base + g * CHUNK, CHUNK)], wsems[s])

        def wait_write(g, s):
            pltpu.make_async_copy(
                bufs.at[s],
                out_hbm.at[pl.ds(base + g * CHUNK, CHUNK)], wsems[s]).wait()

        for s in range(NB):
            gather(s, s)

        def body(t, carry):
            g0 = NB * t
            for s in range(NB):
                g = g0 + s
                wait_gather(g, s)
                write(g, s)
                h = g - DEFER

                @pl.when(jnp.logical_and(h >= 0, h + NB < NCHUNKS))
                def _():
                    hs = (s - DEFER) % NB
                    wait_write(h, hs)
                    gather(h + NB, hs)

            return carry

        lax.fori_loop(0, NROUNDS, body, 0)

        for s in range(NB):
            wait_write(NCHUNKS - NB + s, s)

    return k(pe, idx3)


def kernel(positions, pe):
    idx3 = positions.reshape(NUM_WORKERS, NCHUNKS, CHUNK).astype(jnp.int32)
    out = _sc_gather(pe, idx3)
    return out.reshape(positions.shape + (D_MODEL,))


# speedup vs baseline: 2.4021x; 1.0010x over previous
import functools

import jax
import jax.numpy as jnp
from jax import lax
from jax.experimental import pallas as pl
from jax.experimental.pallas import tpu as pltpu
from jax.experimental.pallas import tpu_sc as plsc

D_MODEL = 1024
MAX_LEN = 8192
B_TOTAL = 4 * 8192
NUM_WORKERS = 32
B_PER_W = B_TOTAL // NUM_WORKERS
NB = 8
CHUNK = 8
DEFER = 2
NCHUNKS = B_PER_W // CHUNK
NROUNDS = NCHUNKS // NB


def _sc_gather(pe, idx3):
    mesh = plsc.VectorSubcoreMesh(core_axis_name="c", subcore_axis_name="s")
    num_cores = mesh.num_cores

    @functools.partial(
        pl.kernel,
        out_type=jax.ShapeDtypeStruct((B_TOTAL, D_MODEL), jnp.float32),
        mesh=mesh,
        scratch_types=[
            pltpu.VMEM((NCHUNKS, CHUNK), jnp.int32),
            pltpu.VMEM((NB, CHUNK, D_MODEL), jnp.float32),
            [pltpu.SemaphoreType.DMA] * NB,
            [pltpu.SemaphoreType.DMA] * NB,
        ],
    )
    def k(pe_hbm, idx_hbm, out_hbm, idx_v, bufs, gsems, wsems):
        wid = lax.axis_index("s") * num_cores + lax.axis_index("c")
        base = wid * B_PER_W
        pltpu.sync_copy(idx_hbm.at[wid], idx_v)

        def gather(g, s):
            return pltpu.async_copy(pe_hbm.at[idx_v.at[g]], bufs.at[s], gsems[s])

        def wait_gather(g, s):
            pltpu.make_async_copy(
                pe_hbm.at[idx_v.at[g]], bufs.at[s], gsems[s]).wait()

        def write(g, s):
            return pltpu.async_copy(
                bufs.at[s], out_hbm.at[pl.ds(base + g * CHUNK, CHUNK)], wsems[s])

        def wait_write(g, s):
            pltpu.make_async_copy(
                bufs.at[s],
                out_hbm.at[pl.ds(base + g * CHUNK, CHUNK)], wsems[s]).wait()

        for s in range(NB):
            gather(s, s)

        def body(t, carry):
            g0 = NB * t
            for s in range(NB):
                g = g0 + s
                wait_gather(g, s)
                write(g, s)
                h = g - DEFER

                @pl.when(jnp.logical_and(h >= 0, h + NB < NCHUNKS))
                def _():
                    hs = (s - DEFER) % NB
                    wait_write(h, hs)
                    gather(h + NB, hs)

            return carry

        lax.fori_loop(0, NROUNDS, body, 0)

        for s in range(NB):
            wait_write(NCHUNKS - NB + s, s)

    return k(pe, idx3)


def kernel(positions, pe):
    idx3 = positions.reshape(NUM_WORKERS, NCHUNKS, CHUNK).astype(jnp.int32)
    out = _sc_gather(pe, idx3)
    return out.reshape(positions.shape + (D_MODEL,))


# speedup vs baseline: 2.4259x; 1.0099x over previous
import functools

import jax
import jax.numpy as jnp
from jax import lax
from jax.experimental import pallas as pl
from jax.experimental.pallas import tpu as pltpu
from jax.experimental.pallas import tpu_sc as plsc

D_MODEL = 1024
MAX_LEN = 8192
BATCH = 4
SEQ = 8192
B_TOTAL = BATCH * SEQ
NUM_WORKERS = 32
B_PER_W = B_TOTAL // NUM_WORKERS
W_PER_BATCH = SEQ // B_PER_W
NB = 8
CHUNK = 8
DEFER = 4
NCHUNKS = B_PER_W // CHUNK
NROUNDS = NCHUNKS // NB


def _sc_gather(pe, positions):
    mesh = plsc.VectorSubcoreMesh(core_axis_name="c", subcore_axis_name="s")
    num_cores = mesh.num_cores

    @functools.partial(
        pl.kernel,
        out_type=jax.ShapeDtypeStruct((BATCH, SEQ, D_MODEL), jnp.float32),
        mesh=mesh,
        scratch_types=[
            pltpu.VMEM((B_PER_W,), jnp.int32),
            pltpu.VMEM((NB, CHUNK, D_MODEL), jnp.float32),
            [pltpu.SemaphoreType.DMA] * NB,
            [pltpu.SemaphoreType.DMA] * NB,
        ],
    )
    def k(pe_hbm, idx_hbm, out_hbm, idx_v, bufs, gsems, wsems):
        wid = lax.axis_index("s") * num_cores + lax.axis_index("c")
        brow = wid // W_PER_BATCH
        boff = (wid % W_PER_BATCH) * B_PER_W
        pltpu.sync_copy(idx_hbm.at[brow, pl.ds(boff, B_PER_W)], idx_v)

        def gather(g, s):
            return pltpu.async_copy(
                pe_hbm.at[idx_v.at[pl.ds(g * CHUNK, CHUNK)]], bufs.at[s],
                gsems[s])

        def wait_gather(g, s):
            pltpu.make_async_copy(
                pe_hbm.at[idx_v.at[pl.ds(g * CHUNK, CHUNK)]], bufs.at[s],
                gsems[s]).wait()

        def write(g, s):
            return pltpu.async_copy(
                bufs.at[s],
                out_hbm.at[brow, pl.ds(boff + g * CHUNK, CHUNK)], wsems[s])

        def wait_write(g, s):
            pltpu.make_async_copy(
                bufs.at[s],
                out_hbm.at[brow, pl.ds(boff + g * CHUNK, CHUNK)],
                wsems[s]).wait()

        for s in range(NB):
            gather(s, s)

        def body(t, carry):
            g0 = NB * t
            for s in range(NB):
                g = g0 + s
                wait_gather(g, s)
                write(g, s)
                h = g - DEFER

                @pl.when(jnp.logical_and(h >= 0, h + NB < NCHUNKS))
                def _():
                    hs = (s - DEFER) % NB
                    wait_write(h, hs)
                    gather(h + NB, hs)

            return carry

        lax.fori_loop(0, NROUNDS, body, 0)

        for s in range(NB):
            wait_write(NCHUNKS - NB + s, s)

    return k(pe, positions)


def kernel(positions, pe):
    return _sc_gather(pe, positions.astype(jnp.int32))


# speedup vs baseline: 2.4276x; 1.0007x over previous
import functools

import jax
import jax.numpy as jnp
from jax import lax
from jax.experimental import pallas as pl
from jax.experimental.pallas import tpu as pltpu
from jax.experimental.pallas import tpu_sc as plsc

D_MODEL = 1024
MAX_LEN = 8192
BATCH = 4
SEQ = 8192
B_TOTAL = BATCH * SEQ
NUM_WORKERS = 32
B_PER_W = B_TOTAL // NUM_WORKERS
W_PER_BATCH = SEQ // B_PER_W
NB = 8
CHUNK = 8
DEFER = 4
NCHUNKS = B_PER_W // CHUNK
NROUNDS = NCHUNKS // NB


def _sc_gather(pe, positions):
    mesh = plsc.VectorSubcoreMesh(core_axis_name="c", subcore_axis_name="s")
    num_cores = mesh.num_cores

    @functools.partial(
        pl.kernel,
        out_type=jax.ShapeDtypeStruct((BATCH, SEQ, D_MODEL), jnp.float32),
        mesh=mesh,
        scratch_types=[
            pltpu.VMEM((B_PER_W,), jnp.int32),
            pltpu.VMEM((NB, CHUNK, D_MODEL), jnp.float32),
            [pltpu.SemaphoreType.DMA] * NB,
            [pltpu.SemaphoreType.DMA] * NB,
            pltpu.SemaphoreType.DMA,
        ],
    )
    def k(pe_hbm, idx_hbm, out_hbm, idx_v, bufs, gsems, wsems, isem):
        wid = lax.axis_index("s") * num_cores + lax.axis_index("c")
        brow = wid // W_PER_BATCH
        boff = (wid % W_PER_BATCH) * B_PER_W
        head = 128
        pltpu.sync_copy(idx_hbm.at[brow, pl.ds(boff, head)],
                        idx_v.at[pl.ds(0, head)])
        tail = pltpu.async_copy(
            idx_hbm.at[brow, pl.ds(boff + head, B_PER_W - head)],
            idx_v.at[pl.ds(head, B_PER_W - head)], isem)

        def gather(g, s):
            return pltpu.async_copy(
                pe_hbm.at[idx_v.at[pl.ds(g * CHUNK, CHUNK)]], bufs.at[s],
                gsems[s])

        def wait_gather(g, s):
            pltpu.make_async_copy(
                pe_hbm.at[idx_v.at[pl.ds(g * CHUNK, CHUNK)]], bufs.at[s],
                gsems[s]).wait()

        def write(g, s):
            return pltpu.async_copy(
                bufs.at[s],
                out_hbm.at[brow, pl.ds(boff + g * CHUNK, CHUNK)], wsems[s])

        def wait_write(g, s):
            pltpu.make_async_copy(
                bufs.at[s],
                out_hbm.at[brow, pl.ds(boff + g * CHUNK, CHUNK)],
                wsems[s]).wait()

        for s in range(NB):
            gather(s, s)
        tail.wait()

        def body(t, carry):
            g0 = NB * t
            for s in range(NB):
                g = g0 + s
                wait_gather(g, s)
                write(g, s)
                h = g - DEFER

                @pl.when(jnp.logical_and(h >= 0, h + NB < NCHUNKS))
                def _():
                    hs = (s - DEFER) % NB
                    wait_write(h, hs)
                    gather(h + NB, hs)

            return carry

        lax.fori_loop(0, NROUNDS, body, 0)

        for s in range(NB):
            wait_write(NCHUNKS - NB + s, s)

    return k(pe, positions)


def kernel(positions, pe):
    return _sc_gather(pe, positions.astype(jnp.int32))
